# baseline (speedup 1.0000x reference)
import jax
import jax.numpy as jnp
from jax import lax
from jax.experimental import pallas as pl
from jax.experimental.pallas import tpu as pltpu

N_FACTORS = 16
B_USERS = 16384
B_ITEMS = 4096
BM = 512
NBLK = B_USERS // BM


def _fused_body(users_s, items_s, uf_any, if_any, o_ref,
                ubuf, vbuf, usem, isem):
    i = pl.program_id(0)

    def _fire_users(blk, buf_slot):
        def ub(j, c):
            idx = users_s[blk * BM + j]
            pltpu.async_copy(uf_any.at[pl.ds(idx, 1), :],
                             ubuf.at[buf_slot, pl.ds(j, 1), :],
                             usem.at[buf_slot])
            return c

        lax.fori_loop(0, BM, ub, 0, unroll=8)

    @pl.when(i == 0)
    def _prologue():
        def ib(j, c):
            idx = items_s[j]
            pltpu.async_copy(if_any.at[pl.ds(idx, 1), :],
                             vbuf.at[pl.ds(j, 1), :], isem)
            return c

        lax.fori_loop(0, B_ITEMS, ib, 0, unroll=8)
        _fire_users(0, 0)

    @pl.when(i < NBLK - 1)
    def _fire_next():
        _fire_users(i + 1, (i + 1) % 2)

    @pl.when(i == 0)
    def _wait_items():
        pltpu.make_async_copy(if_any.at[pl.ds(0, B_ITEMS), :], vbuf,
                              isem).wait()

    def _compute(slot):
        pltpu.make_async_copy(uf_any.at[pl.ds(0, BM), :],
                              ubuf.at[slot], usem.at[slot]).wait()
        o_ref[...] = lax.dot_general(ubuf[slot], vbuf[...],
                                     (((1,), (1,)), ((), ())),
                                     preferred_element_type=jnp.float32)

    @pl.when(i % 2 == 0)
    def _c0():
        _compute(0)

    @pl.when(i % 2 == 1)
    def _c1():
        _compute(1)


def kernel(users, items, user_factors, item_factors):
    grid_spec = pltpu.PrefetchScalarGridSpec(
        num_scalar_prefetch=2,
        grid=(NBLK,),
        in_specs=[
            pl.BlockSpec(memory_space=pl.ANY),
            pl.BlockSpec(memory_space=pl.ANY),
        ],
        out_specs=pl.BlockSpec((BM, B_ITEMS), lambda i, u_s, i_s: (i, 0)),
        scratch_shapes=[
            pltpu.VMEM((2, BM, N_FACTORS), jnp.float32),
            pltpu.VMEM((B_ITEMS, N_FACTORS), jnp.float32),
            pltpu.SemaphoreType.DMA((2,)),
            pltpu.SemaphoreType.DMA,
        ],
    )
    return pl.pallas_call(
        _fused_body,
        grid_spec=grid_spec,
        out_shape=jax.ShapeDtypeStruct((B_USERS, B_ITEMS), jnp.float32),
    )(users.astype(jnp.int32), items.astype(jnp.int32),
      user_factors, item_factors)
